# baseline (speedup 1.0000x reference)
import functools

import jax
import jax.numpy as jnp
from jax import lax
from jax.experimental import pallas as pl
from jax.experimental.pallas import tpu as pltpu
from jax.experimental.pallas import tpu_sc as plsc

NSUB = 16
NCORE = 2
CH = 128
LANES = 16
TRASH_PAD = 16


def _ceil_to(a, m):
    return (a + m - 1) // m * m


def _build_prep(NP, EP, EWP):
    EW = EP // NSUB
    NR = NP // NSUB
    HNP = NP // NCORE
    mesh = plsc.VectorSubcoreMesh(core_axis_name="c", subcore_axis_name="s")

    @functools.partial(
        pl.kernel,
        out_type=(jax.ShapeDtypeStruct((NP,), jnp.float32),
                  jax.ShapeDtypeStruct((NP,), jnp.float32),
                  jax.ShapeDtypeStruct((NCORE, NSUB, EWP), jnp.int32),
                  jax.ShapeDtypeStruct((NCORE, NSUB, EWP), jnp.int32),
                  jax.ShapeDtypeStruct((NCORE, NSUB, LANES), jnp.int32)),
        mesh=mesh,
        compiler_params=pltpu.CompilerParams(needs_layout_passes=False),
        scratch_types=[
            pltpu.VMEM((EWP,), jnp.int32),
            pltpu.VMEM((EWP,), jnp.int32),
            pltpu.VMEM((EWP,), jnp.int32),
            pltpu.VMEM((EWP,), jnp.int32),
            pltpu.VMEM((LANES,), jnp.int32),
            pltpu.VMEM((NP,), jnp.float32),
            pltpu.VMEM_SHARED((NSUB, NP), jnp.float32),
            pltpu.VMEM((NR,), jnp.float32),
            pltpu.VMEM((NR,), jnp.float32),
            pltpu.VMEM((NR,), jnp.float32),
        ],
    )
    def prep(src_hbm, dst_hbm, dinv_hbm, dinv2_hbm, ps_hbm, pd_hbm, cnt_hbm,
             src_v, dst_v, s1_v, d1_v, cnt_v, degloc, shp, accs, tmps, d2s):
        wid = lax.axis_index("s")
        core = lax.axis_index("c")
        zero16 = jnp.zeros((LANES,), jnp.float32)

        pltpu.sync_copy(src_hbm.at[pl.ds(wid * EW, EW)],
                        src_v.at[pl.ds(0, EW)])
        pltpu.sync_copy(dst_hbm.at[pl.ds(wid * EW, EW)],
                        dst_v.at[pl.ds(0, EW)])

        def zloc(i, _):
            degloc[pl.ds(i * LANES, LANES)] = zero16
            return 0
        lax.fori_loop(0, NP // LANES, zloc, 0)

        ones = jnp.ones((LANES,), jnp.float32)

        def scat(i, _):
            idx = dst_v[pl.ds(i * LANES, LANES)]
            plsc.addupdate_scatter(degloc, [idx], ones)
            return 0
        lax.fori_loop(0, EW // LANES, scat, 0)

        padsrc = jnp.full((LANES,), NP - 1, jnp.int32)
        padded = jnp.full((LANES,), HNP, jnp.int32)

        def part(i, c01):
            c0, c1 = c01
            s = src_v[pl.ds(i * LANES, LANES)]
            d = dst_v[pl.ds(i * LANES, LANES)]
            m0 = d < HNP
            m1 = jnp.logical_not(m0)
            plsc.store_compressed(dst_v.at[pl.ds(c0, LANES)], d, mask=m0)
            plsc.store_compressed(src_v.at[pl.ds(c0, LANES)], s, mask=m0)
            plsc.store_compressed(d1_v.at[pl.ds(c1, LANES)], d - HNP, mask=m1)
            plsc.store_compressed(s1_v.at[pl.ds(c1, LANES)], s, mask=m1)
            n0 = jnp.max(plsc.all_reduce_population_count(m0))
            return (c0 + n0, c1 + (LANES - n0))
        c0, c1 = lax.fori_loop(0, EW // LANES, part, (0, 0))

        def pad0(k, _):
            src_v[pl.ds(c0 + k * LANES, LANES)] = padsrc
            dst_v[pl.ds(c0 + k * LANES, LANES)] = padded
            return 0
        lax.fori_loop(0, CH // LANES, pad0, 0)

        def pad1(k, _):
            s1_v[pl.ds(c1 + k * LANES, LANES)] = padsrc
            d1_v[pl.ds(c1 + k * LANES, LANES)] = padded
            return 0
        lax.fori_loop(0, CH // LANES, pad1, 0)

        pltpu.sync_copy(src_v, ps_hbm.at[0, wid])
        pltpu.sync_copy(dst_v, pd_hbm.at[0, wid])
        pltpu.sync_copy(s1_v, ps_hbm.at[1, wid])
        pltpu.sync_copy(d1_v, pd_hbm.at[1, wid])
        nc0 = (c0 + CH - 1) // CH
        nc1 = (c1 + CH - 1) // CH
        cnt_v[...] = jnp.zeros((LANES,), jnp.int32) + nc0
        pltpu.sync_copy(cnt_v, cnt_hbm.at[0, wid])
        cnt_v[...] = jnp.zeros((LANES,), jnp.int32) + nc1
        pltpu.sync_copy(cnt_v, cnt_hbm.at[1, wid])

        pltpu.sync_copy(degloc, shp.at[wid])
        plsc.subcore_barrier()

        def zacc(i, _):
            accs[pl.ds(i * LANES, LANES)] = zero16
            return 0
        lax.fori_loop(0, NR // LANES, zacc, 0)

        for j in range(NSUB):
            pltpu.sync_copy(shp.at[j, pl.ds(wid * NR, NR)], tmps)

            def addv(i, _):
                s = pl.ds(i * LANES, LANES)
                accs[s] = accs[s] + tmps[s]
                return 0
            lax.fori_loop(0, NR // LANES, addv, 0)

        def elemw(i, _):
            s = pl.ds(i * LANES, LANES)
            d = jnp.maximum(accs[s], 1.0)
            d2s[s] = 1.0 / d
            ii = plsc.bitcast(d, jnp.int32)
            ii = jnp.int32(0x5F3759DF) - (ii >> 1)
            y = plsc.bitcast(ii, jnp.float32)
            for _ in range(3):
                y = y * (1.5 - 0.5 * d * y * y)
            accs[s] = y
            return 0
        lax.fori_loop(0, NR // LANES, elemw, 0)

        @pl.when(core == 0)
        def _():
            pltpu.sync_copy(accs, dinv_hbm.at[pl.ds(wid * NR, NR)])
            pltpu.sync_copy(d2s, dinv2_hbm.at[pl.ds(wid * NR, NR)])

    return prep


def _build_adj_apply(NP, EWP, D, n_comp):
    NCHC = EWP // CH
    HNP = NP // NCORE
    NH = HNP + TRASH_PAD
    NR = HNP // NSUB
    mesh = plsc.VectorSubcoreMesh(core_axis_name="c", subcore_axis_name="s")

    @functools.partial(
        pl.kernel,
        out_type=tuple(jax.ShapeDtypeStruct((NP, D), jnp.float32)
                       for _ in range(n_comp)),
        mesh=mesh,
        compiler_params=pltpu.CompilerParams(needs_layout_passes=False),
        scratch_types=[
            pltpu.VMEM((NCHC, CH), jnp.int32),
            pltpu.VMEM((NCHC, CH), jnp.int32),
            pltpu.VMEM((LANES,), jnp.int32),
            pltpu.VMEM((CH, D), jnp.float32),
            pltpu.VMEM((CH, D), jnp.float32),
            pltpu.VMEM_SHARED((NH, D), jnp.float32),
        ] + [pltpu.SemaphoreType.DMA] * 4,
    )
    def adj(*refs):
        v_hbms = refs[4:4 + n_comp]
        s_os = refs[4 + n_comp:4 + 2 * n_comp]
        src_hbm, dloc_hbm, cnt_hbm, z_hbm = refs[:4]
        (src_v, dst_v, cnt_v, rows0, rows1, acc,
         g0, g1, s0, s1) = refs[4 + 2 * n_comp:]
        bufs = (rows0, rows1)
        gsems = (g0, g1)
        ssems = (s0, s1)
        S = 2
        AG = 1
        AS = 1
        wid = lax.axis_index("s")
        core = lax.axis_index("c")

        pltpu.sync_copy(src_hbm.at[core, wid], src_v)
        pltpu.sync_copy(dloc_hbm.at[core, wid], dst_v)
        pltpu.sync_copy(cnt_hbm.at[core, wid], cnt_v)
        ncw = jnp.maximum(jnp.max(cnt_v[...]), 1)

        def comp(v_hbm, s_hbm):
            def zacc(k, _):
                pltpu.sync_copy(z_hbm, acc.at[pl.ds(wid * NR + k * 32, 32)])
                return 0
            lax.fori_loop(0, NR // 32, zacc, 0)
            plsc.subcore_barrier()

            for c in range(AG):
                pltpu.async_copy(v_hbm.at[src_v.at[c]], bufs[c % S],
                                 gsems[c % S])

            def group(q, _):
                for bslot in range(S):
                    c = q * S + bslot

                    @pl.when(c < ncw)
                    def _(c=c, bslot=bslot):
                        pltpu.make_async_copy(
                            v_hbm.at[src_v.at[c]], bufs[bslot],
                            gsems[bslot]).wait()
                        pltpu.async_copy(
                            bufs[bslot], acc.at[dst_v.at[c]], ssems[bslot],
                            add=True)
                        bd = (bslot - AS) % S

                        @pl.when(c >= AS)
                        def _():
                            pltpu.make_async_copy(
                                bufs[bd], acc.at[dst_v.at[c - AS]],
                                ssems[bd]).wait()
                        bg = (bslot + AG) % S

                        @pl.when(c + AG < ncw)
                        def _():
                            pltpu.async_copy(
                                v_hbm.at[src_v.at[c + AG]], bufs[bg],
                                gsems[bg])
                return 0
            lax.fori_loop(0, (ncw + S - 1) // S, group, 0)
            last = ncw - 1
            for k in range(S):
                @pl.when(last % S == k)
                def _(k=k):
                    pltpu.make_async_copy(
                        bufs[k], acc.at[dst_v.at[last]], ssems[k]).wait()
            for c in range(1, AG):
                @pl.when(ncw <= c)
                def _(c=c):
                    pltpu.make_async_copy(
                        v_hbm.at[src_v.at[c]], bufs[c % S],
                        gsems[c % S]).wait()
            plsc.subcore_barrier()

            def wout(k, _):
                loc = pl.ds(wid * NR + k * 64, 64)
                glob = pl.ds(core * HNP + wid * NR + k * 64, 64)
                pltpu.sync_copy(acc.at[loc], s_hbm.at[glob])
                return 0
            lax.fori_loop(0, NR // 64, wout, 0)
            plsc.subcore_barrier()

        for v_hbm, s_hbm in zip(v_hbms, s_os):
            comp(v_hbm, s_hbm)

    return adj


def _build_cmatmul(NP, D, first, resid, BN=1024):
    grid = (NP // BN,)
    rows = pl.BlockSpec((BN, D), lambda i: (i, 0))
    wspec = pl.BlockSpec((D, D), lambda i: (0, 0))
    cols = pl.BlockSpec((BN, 1), lambda i: (i, 0))
    outs = [jax.ShapeDtypeStruct((NP, D), jnp.float32)] * 4

    if first:
        def body(xr_r, wr_r, wi_r, dv_r, yr_o, yi_o, vr_o, vi_o):
            xr = xr_r[...]
            hr = jnp.dot(xr, wr_r[...], preferred_element_type=jnp.float32)
            hi = jnp.dot(xr, wi_r[...], preferred_element_type=jnp.float32)
            dv = dv_r[...]
            yr_o[...] = hr
            yi_o[...] = hi
            vr_o[...] = hr * dv
            vi_o[...] = hi * dv
        in_specs = [rows, wspec, wspec, cols]
    else:
        def body(xr_r, xi_r, wr_r, wi_r, dv_r, yr_o, yi_o, vr_o, vi_o):
            xr = xr_r[...]
            xi = xi_r[...]
            wr = wr_r[...]
            wi = wi_r[...]
            hr = (jnp.dot(xr, wr, preferred_element_type=jnp.float32)
                  - jnp.dot(xi, wi, preferred_element_type=jnp.float32))
            hi = (jnp.dot(xr, wi, preferred_element_type=jnp.float32)
                  + jnp.dot(xi, wr, preferred_element_type=jnp.float32))
            dv = dv_r[...]
            yr_o[...] = hr + (xr if resid else 0.0)
            yi_o[...] = hi + (xi if resid else 0.0)
            vr_o[...] = hr * dv
            vi_o[...] = hi * dv
        in_specs = [rows, rows, wspec, wspec, cols]

    return pl.pallas_call(
        body, grid=grid, in_specs=in_specs,
        out_specs=[rows] * 4, out_shape=outs)


def _build_stepelem(NP, D, inv_t, last, BN=1024):
    grid = (NP // BN,)
    rows = pl.BlockSpec((BN, D), lambda i: (i, 0))
    cols = pl.BlockSpec((BN, 1), lambda i: (i, 0))
    n_out = 2 if last else 4
    outs = [jax.ShapeDtypeStruct((NP, D), jnp.float32)] * n_out

    def body(sr_r, si_r, yr_r, yi_r, dv_r, dv2_r, *out_refs):
        sr = sr_r[...]
        si = si_r[...]
        dv = dv_r[...] * inv_t
        out_refs[0][...] = yr_r[...] - si * dv
        out_refs[1][...] = yi_r[...] + sr * dv
        if not last:
            dv2 = dv2_r[...] * inv_t
            out_refs[2][...] = -si * dv2
            out_refs[3][...] = sr * dv2

    return pl.pallas_call(
        body, grid=grid,
        in_specs=[rows, rows, rows, rows, cols, cols],
        out_specs=[rows] * n_out, out_shape=outs)


def _build_sage(NP, D, BN=1024):
    grid = (NP // BN,)
    rows = pl.BlockSpec((BN, D), lambda i: (i, 0))
    wspec = pl.BlockSpec((D, D), lambda i: (0, 0))
    cols = pl.BlockSpec((BN, 1), lambda i: (i, 0))
    bspec = pl.BlockSpec((1, D), lambda i: (0, 0))

    def body(p_r, y_r, dv2_r, wl_r, wr_r, b_r, o_r):
        mean = p_r[...] * dv2_r[...]
        o_r[...] = (jnp.dot(mean, wl_r[...], preferred_element_type=jnp.float32)
                    + jnp.dot(y_r[...], wr_r[...], preferred_element_type=jnp.float32)
                    + b_r[...])

    return pl.pallas_call(
        body, grid=grid,
        in_specs=[rows, rows, cols, wspec, wspec, bspec],
        out_specs=rows, out_shape=jax.ShapeDtypeStruct((NP, D), jnp.float32))


def kernel(x, edge_index, Wre0, Wre1, Wre2, Wre3, Wre4,
           Wim0, Wim1, Wim2, Wim3, Wim4, Wl, Wr, b):
    N, D = x.shape
    E = edge_index.shape[1]
    T = 4
    NP = _ceil_to(N, NCORE * NSUB * 64)
    EP = _ceil_to(E, NSUB * CH)
    while (EP // NSUB) % LANES:
        EP += NSUB * CH
    EWP = _ceil_to(EP // NSUB + CH, CH)

    src = edge_index[0]
    dst = edge_index[1]
    if EP > E:
        pad = jnp.full((EP - E,), N, jnp.int32)
        src = jnp.concatenate([src, pad])
        dst = jnp.concatenate([dst, pad])
    xp = jnp.pad(x, ((0, NP - N), (0, 0)))

    prep = _build_prep(NP, EP, EWP)
    adj = _build_adj_apply(NP, EWP, D, n_comp=2)
    aggk = _build_adj_apply(NP, EWP, D, n_comp=1)

    dinv, dinv2, psrc, pdst, cnts = prep(src, dst)
    zrows = jnp.zeros((32, D), jnp.float32)
    psrc = psrc.reshape(NCORE, NSUB, EWP // CH, CH)
    pdst = pdst.reshape(NCORE, NSUB, EWP // CH, CH)
    dinv_c = dinv.reshape(NP, 1)
    dinv2_c = dinv2.reshape(NP, 1)

    Wres = [Wre0, Wre1, Wre2, Wre3, Wre4]
    Wims = [Wim0, Wim1, Wim2, Wim3, Wim4]
    NL = len(Wres)

    cm_first = _build_cmatmul(NP, D, first=True, resid=False)
    cm_rest = _build_cmatmul(NP, D, first=False, resid=True)
    steps = [_build_stepelem(NP, D, 1.0 / t, last=(t == T))
             for t in range(1, T + 1)]

    yr = yi = None
    for l in range(NL):
        if l == 0:
            yr, yi, vr, vi = cm_first(xp, Wres[0], Wims[0], dinv_c)
        else:
            yr, yi, vr, vi = cm_rest(yr, yi, Wres[l], Wims[l], dinv_c)
        for t in range(1, T + 1):
            sr, si = adj(psrc, pdst, cnts, zrows, vr, vi)
            if t < T:
                yr, yi, vr, vi = steps[t - 1](sr, si, yr, yi, dinv_c, dinv2_c)
            else:
                yr, yi = steps[t - 1](sr, si, yr, yi, dinv_c, dinv2_c)

    (p,) = aggk(psrc, pdst, cnts, zrows, yr)
    out = _build_sage(NP, D)(p, yr, dinv2_c, Wl, Wr, b.reshape(1, D))
    return out[:N]


# speedup vs baseline: 1.0380x; 1.0380x over previous
import functools

import jax
import jax.numpy as jnp
from jax import lax
from jax.experimental import pallas as pl
from jax.experimental.pallas import tpu as pltpu
from jax.experimental.pallas import tpu_sc as plsc

NSUB = 16
NCORE = 2
CH = 128
LANES = 16
TRASH_PAD = 16


def _ceil_to(a, m):
    return (a + m - 1) // m * m


def _build_prep(NP, EP, EWP):
    EW = EP // NSUB
    NR = NP // NSUB
    HNP = NP // NCORE
    mesh = plsc.VectorSubcoreMesh(core_axis_name="c", subcore_axis_name="s")

    @functools.partial(
        pl.kernel,
        out_type=(jax.ShapeDtypeStruct((NP,), jnp.float32),
                  jax.ShapeDtypeStruct((NP,), jnp.float32),
                  jax.ShapeDtypeStruct((NCORE, NSUB, EWP), jnp.int32),
                  jax.ShapeDtypeStruct((NCORE, NSUB, EWP), jnp.int32),
                  jax.ShapeDtypeStruct((NCORE, NSUB, LANES), jnp.int32)),
        mesh=mesh,
        compiler_params=pltpu.CompilerParams(needs_layout_passes=False),
        scratch_types=[
            pltpu.VMEM((EWP,), jnp.int32),
            pltpu.VMEM((EWP,), jnp.int32),
            pltpu.VMEM((EWP,), jnp.int32),
            pltpu.VMEM((EWP,), jnp.int32),
            pltpu.VMEM((LANES,), jnp.int32),
            pltpu.VMEM((NP,), jnp.float32),
            pltpu.VMEM_SHARED((NSUB, NP), jnp.float32),
            pltpu.VMEM((NR,), jnp.float32),
            pltpu.VMEM((NR,), jnp.float32),
            pltpu.VMEM((NR,), jnp.float32),
        ],
    )
    def prep(src_hbm, dst_hbm, dinv_hbm, dinv2_hbm, ps_hbm, pd_hbm, cnt_hbm,
             src_v, dst_v, s1_v, d1_v, cnt_v, degloc, shp, accs, tmps, d2s):
        wid = lax.axis_index("s")
        core = lax.axis_index("c")
        zero16 = jnp.zeros((LANES,), jnp.float32)

        pltpu.sync_copy(src_hbm.at[pl.ds(wid * EW, EW)],
                        src_v.at[pl.ds(0, EW)])
        pltpu.sync_copy(dst_hbm.at[pl.ds(wid * EW, EW)],
                        dst_v.at[pl.ds(0, EW)])

        def zloc(i, _):
            degloc[pl.ds(i * LANES, LANES)] = zero16
            return 0
        lax.fori_loop(0, NP // LANES, zloc, 0)

        ones = jnp.ones((LANES,), jnp.float32)

        def scat(i, _):
            idx = dst_v[pl.ds(i * LANES, LANES)]
            plsc.addupdate_scatter(degloc, [idx], ones)
            return 0
        lax.fori_loop(0, EW // LANES, scat, 0)

        padsrc = jnp.full((LANES,), NP - 1, jnp.int32)
        padded = jnp.full((LANES,), HNP, jnp.int32)

        def part(i, c01):
            c0, c1 = c01
            s = src_v[pl.ds(i * LANES, LANES)]
            d = dst_v[pl.ds(i * LANES, LANES)]
            m0 = d < HNP
            m1 = jnp.logical_not(m0)
            plsc.store_compressed(dst_v.at[pl.ds(c0, LANES)], d, mask=m0)
            plsc.store_compressed(src_v.at[pl.ds(c0, LANES)], s, mask=m0)
            plsc.store_compressed(d1_v.at[pl.ds(c1, LANES)], d - HNP, mask=m1)
            plsc.store_compressed(s1_v.at[pl.ds(c1, LANES)], s, mask=m1)
            n0 = jnp.max(plsc.all_reduce_population_count(m0))
            return (c0 + n0, c1 + (LANES - n0))
        c0, c1 = lax.fori_loop(0, EW // LANES, part, (0, 0))

        def pad0(k, _):
            src_v[pl.ds(c0 + k * LANES, LANES)] = padsrc
            dst_v[pl.ds(c0 + k * LANES, LANES)] = padded
            return 0
        lax.fori_loop(0, CH // LANES, pad0, 0)

        def pad1(k, _):
            s1_v[pl.ds(c1 + k * LANES, LANES)] = padsrc
            d1_v[pl.ds(c1 + k * LANES, LANES)] = padded
            return 0
        lax.fori_loop(0, CH // LANES, pad1, 0)

        pltpu.sync_copy(src_v, ps_hbm.at[0, wid])
        pltpu.sync_copy(dst_v, pd_hbm.at[0, wid])
        pltpu.sync_copy(s1_v, ps_hbm.at[1, wid])
        pltpu.sync_copy(d1_v, pd_hbm.at[1, wid])
        nc0 = (c0 + CH - 1) // CH
        nc1 = (c1 + CH - 1) // CH
        cnt_v[...] = jnp.zeros((LANES,), jnp.int32) + nc0
        pltpu.sync_copy(cnt_v, cnt_hbm.at[0, wid])
        cnt_v[...] = jnp.zeros((LANES,), jnp.int32) + nc1
        pltpu.sync_copy(cnt_v, cnt_hbm.at[1, wid])

        pltpu.sync_copy(degloc, shp.at[wid])
        plsc.subcore_barrier()

        def zacc(i, _):
            accs[pl.ds(i * LANES, LANES)] = zero16
            return 0
        lax.fori_loop(0, NR // LANES, zacc, 0)

        for j in range(NSUB):
            pltpu.sync_copy(shp.at[j, pl.ds(wid * NR, NR)], tmps)

            def addv(i, _):
                s = pl.ds(i * LANES, LANES)
                accs[s] = accs[s] + tmps[s]
                return 0
            lax.fori_loop(0, NR // LANES, addv, 0)

        def elemw(i, _):
            s = pl.ds(i * LANES, LANES)
            d = jnp.maximum(accs[s], 1.0)
            d2s[s] = 1.0 / d
            ii = plsc.bitcast(d, jnp.int32)
            ii = jnp.int32(0x5F3759DF) - (ii >> 1)
            y = plsc.bitcast(ii, jnp.float32)
            for _ in range(3):
                y = y * (1.5 - 0.5 * d * y * y)
            accs[s] = y
            return 0
        lax.fori_loop(0, NR // LANES, elemw, 0)

        @pl.when(core == 0)
        def _():
            pltpu.sync_copy(accs, dinv_hbm.at[pl.ds(wid * NR, NR)])
            pltpu.sync_copy(d2s, dinv2_hbm.at[pl.ds(wid * NR, NR)])

    return prep


def _build_adj_apply(NP, EWP, D, n_comp):
    NCHC = EWP // CH
    HNP = NP // NCORE
    NH = HNP + TRASH_PAD
    NR = HNP // NSUB
    mesh = plsc.VectorSubcoreMesh(core_axis_name="c", subcore_axis_name="s")

    @functools.partial(
        pl.kernel,
        out_type=tuple(jax.ShapeDtypeStruct((NP, D), jnp.float32)
                       for _ in range(n_comp)),
        mesh=mesh,
        compiler_params=pltpu.CompilerParams(needs_layout_passes=False),
        scratch_types=[
            pltpu.VMEM((NCHC, CH), jnp.int32),
            pltpu.VMEM((NCHC, CH), jnp.int32),
            pltpu.VMEM((LANES,), jnp.int32),
            pltpu.VMEM((CH, D), jnp.float32),
            pltpu.VMEM((CH, D), jnp.float32),
            pltpu.VMEM_SHARED((NH, D), jnp.float32),
            pltpu.VMEM((32, D), jnp.float32),
        ] + [pltpu.SemaphoreType.DMA] * 4,
    )
    def adj(*refs):
        v_hbms = refs[4:4 + n_comp]
        s_os = refs[4 + n_comp:4 + 2 * n_comp]
        src_hbm, dloc_hbm, cnt_hbm, z_hbm = refs[:4]
        (src_v, dst_v, cnt_v, rows0, rows1, acc, zbuf,
         g0, g1, s0, s1) = refs[4 + 2 * n_comp:]
        bufs = (rows0, rows1)
        gsems = (g0, g1)
        ssems = (s0, s1)
        S = 2
        AG = 1
        AS = 1
        wid = lax.axis_index("s")
        core = lax.axis_index("c")

        zero16 = jnp.zeros((LANES,), jnp.float32)

        def zz(i, _):
            for f in range(D // LANES):
                zbuf[i, pl.ds(f * LANES, LANES)] = zero16
            return 0
        lax.fori_loop(0, 32, zz, 0)

        pltpu.sync_copy(src_hbm.at[core, wid], src_v)
        pltpu.sync_copy(dloc_hbm.at[core, wid], dst_v)
        pltpu.sync_copy(cnt_hbm.at[core, wid], cnt_v)
        ncw = jnp.maximum(jnp.max(cnt_v[...]), 1)

        def comp(v_hbm, s_hbm):
            def zacc(k, _):
                pltpu.sync_copy(zbuf, acc.at[pl.ds(wid * NR + k * 32, 32)])
                return 0
            lax.fori_loop(0, NR // 32, zacc, 0)
            plsc.subcore_barrier()

            for c in range(AG):
                pltpu.async_copy(v_hbm.at[src_v.at[c]], bufs[c % S],
                                 gsems[c % S])

            def group(q, _):
                for bslot in range(S):
                    c = q * S + bslot

                    @pl.when(c < ncw)
                    def _(c=c, bslot=bslot):
                        pltpu.make_async_copy(
                            v_hbm.at[src_v.at[c]], bufs[bslot],
                            gsems[bslot]).wait()
                        pltpu.async_copy(
                            bufs[bslot], acc.at[dst_v.at[c]], ssems[bslot],
                            add=True)
                        bd = (bslot - AS) % S

                        @pl.when(c >= AS)
                        def _():
                            pltpu.make_async_copy(
                                bufs[bd], acc.at[dst_v.at[c - AS]],
                                ssems[bd]).wait()
                        bg = (bslot + AG) % S

                        @pl.when(c + AG < ncw)
                        def _():
                            pltpu.async_copy(
                                v_hbm.at[src_v.at[c + AG]], bufs[bg],
                                gsems[bg])
                return 0
            lax.fori_loop(0, (ncw + S - 1) // S, group, 0)
            last = ncw - 1
            for k in range(S):
                @pl.when(last % S == k)
                def _(k=k):
                    pltpu.make_async_copy(
                        bufs[k], acc.at[dst_v.at[last]], ssems[k]).wait()
            for c in range(1, AG):
                @pl.when(ncw <= c)
                def _(c=c):
                    pltpu.make_async_copy(
                        v_hbm.at[src_v.at[c]], bufs[c % S],
                        gsems[c % S]).wait()
            plsc.subcore_barrier()

            def wout(k, _):
                loc = pl.ds(wid * NR + k * 64, 64)
                glob = pl.ds(core * HNP + wid * NR + k * 64, 64)
                pltpu.sync_copy(acc.at[loc], s_hbm.at[glob])
                return 0
            lax.fori_loop(0, NR // 64, wout, 0)
            plsc.subcore_barrier()

        for v_hbm, s_hbm in zip(v_hbms, s_os):
            comp(v_hbm, s_hbm)

    return adj


def _build_cmatmul(NP, D, first, resid, BN=1024):
    grid = (NP // BN,)
    rows = pl.BlockSpec((BN, D), lambda i: (i, 0))
    wspec = pl.BlockSpec((D, D), lambda i: (0, 0))
    cols = pl.BlockSpec((BN, 1), lambda i: (i, 0))
    outs = [jax.ShapeDtypeStruct((NP, D), jnp.float32)] * 4

    if first:
        def body(xr_r, wr_r, wi_r, dv_r, yr_o, yi_o, vr_o, vi_o):
            xr = xr_r[...]
            hr = jnp.dot(xr, wr_r[...], preferred_element_type=jnp.float32)
            hi = jnp.dot(xr, wi_r[...], preferred_element_type=jnp.float32)
            dv = dv_r[...]
            yr_o[...] = hr
            yi_o[...] = hi
            vr_o[...] = hr * dv
            vi_o[...] = hi * dv
        in_specs = [rows, wspec, wspec, cols]
    else:
        def body(xr_r, xi_r, wr_r, wi_r, dv_r, yr_o, yi_o, vr_o, vi_o):
            xr = xr_r[...]
            xi = xi_r[...]
            wr = wr_r[...]
            wi = wi_r[...]
            hr = (jnp.dot(xr, wr, preferred_element_type=jnp.float32)
                  - jnp.dot(xi, wi, preferred_element_type=jnp.float32))
            hi = (jnp.dot(xr, wi, preferred_element_type=jnp.float32)
                  + jnp.dot(xi, wr, preferred_element_type=jnp.float32))
            dv = dv_r[...]
            yr_o[...] = hr + (xr if resid else 0.0)
            yi_o[...] = hi + (xi if resid else 0.0)
            vr_o[...] = hr * dv
            vi_o[...] = hi * dv
        in_specs = [rows, rows, wspec, wspec, cols]

    return pl.pallas_call(
        body, grid=grid, in_specs=in_specs,
        out_specs=[rows] * 4, out_shape=outs)


def _build_stepelem(NP, D, inv_t, last, BN=1024):
    grid = (NP // BN,)
    rows = pl.BlockSpec((BN, D), lambda i: (i, 0))
    cols = pl.BlockSpec((BN, 1), lambda i: (i, 0))
    n_out = 2 if last else 4
    outs = [jax.ShapeDtypeStruct((NP, D), jnp.float32)] * n_out

    def body(sr_r, si_r, yr_r, yi_r, dv_r, dv2_r, *out_refs):
        sr = sr_r[...]
        si = si_r[...]
        dv = dv_r[...] * inv_t
        out_refs[0][...] = yr_r[...] - si * dv
        out_refs[1][...] = yi_r[...] + sr * dv
        if not last:
            dv2 = dv2_r[...] * inv_t
            out_refs[2][...] = -si * dv2
            out_refs[3][...] = sr * dv2

    return pl.pallas_call(
        body, grid=grid,
        in_specs=[rows, rows, rows, rows, cols, cols],
        out_specs=[rows] * n_out, out_shape=outs)


def _build_sage(NP, D, BN=1024):
    grid = (NP // BN,)
    rows = pl.BlockSpec((BN, D), lambda i: (i, 0))
    wspec = pl.BlockSpec((D, D), lambda i: (0, 0))
    cols = pl.BlockSpec((BN, 1), lambda i: (i, 0))
    bspec = pl.BlockSpec((1, D), lambda i: (0, 0))

    def body(p_r, y_r, dv2_r, wl_r, wr_r, b_r, o_r):
        mean = p_r[...] * dv2_r[...]
        o_r[...] = (jnp.dot(mean, wl_r[...], preferred_element_type=jnp.float32)
                    + jnp.dot(y_r[...], wr_r[...], preferred_element_type=jnp.float32)
                    + b_r[...])

    return pl.pallas_call(
        body, grid=grid,
        in_specs=[rows, rows, cols, wspec, wspec, bspec],
        out_specs=rows, out_shape=jax.ShapeDtypeStruct((NP, D), jnp.float32))


def kernel(x, edge_index, Wre0, Wre1, Wre2, Wre3, Wre4,
           Wim0, Wim1, Wim2, Wim3, Wim4, Wl, Wr, b):
    N, D = x.shape
    E = edge_index.shape[1]
    T = 4
    NP = _ceil_to(N, NCORE * NSUB * 64)
    EP = _ceil_to(E, NSUB * CH)
    while (EP // NSUB) % LANES:
        EP += NSUB * CH
    EWP = _ceil_to(EP // NSUB + CH, CH)

    src = edge_index[0]
    dst = edge_index[1]
    if EP > E:
        pad = jnp.full((EP - E,), N, jnp.int32)
        src = jnp.concatenate([src, pad])
        dst = jnp.concatenate([dst, pad])
    xp = jnp.pad(x, ((0, NP - N), (0, 0)))

    prep = _build_prep(NP, EP, EWP)
    adj = _build_adj_apply(NP, EWP, D, n_comp=2)
    aggk = _build_adj_apply(NP, EWP, D, n_comp=1)

    dinv, dinv2, psrc, pdst, cnts = prep(src, dst)
    zrows = jnp.zeros((32, D), jnp.float32)
    psrc = psrc.reshape(NCORE, NSUB, EWP // CH, CH)
    pdst = pdst.reshape(NCORE, NSUB, EWP // CH, CH)
    dinv_c = dinv.reshape(NP, 1)
    dinv2_c = dinv2.reshape(NP, 1)

    Wres = [Wre0, Wre1, Wre2, Wre3, Wre4]
    Wims = [Wim0, Wim1, Wim2, Wim3, Wim4]
    NL = len(Wres)

    cm_first = _build_cmatmul(NP, D, first=True, resid=False)
    cm_rest = _build_cmatmul(NP, D, first=False, resid=True)
    steps = [_build_stepelem(NP, D, 1.0 / t, last=(t == T))
             for t in range(1, T + 1)]

    yr = yi = None
    for l in range(NL):
        if l == 0:
            yr, yi, vr, vi = cm_first(xp, Wres[0], Wims[0], dinv_c)
        else:
            yr, yi, vr, vi = cm_rest(yr, yi, Wres[l], Wims[l], dinv_c)
        for t in range(1, T + 1):
            sr, si = adj(psrc, pdst, cnts, zrows, vr, vi)
            if t < T:
                yr, yi, vr, vi = steps[t - 1](sr, si, yr, yi, dinv_c, dinv2_c)
            else:
                yr, yi = steps[t - 1](sr, si, yr, yi, dinv_c, dinv2_c)

    (p,) = aggk(psrc, pdst, cnts, zrows, yr)
    out = _build_sage(NP, D)(p, yr, dinv2_c, Wl, Wr, b.reshape(1, D))
    return out[:N]
